# baseline (speedup 1.0000x reference)
import functools

import jax
import jax.numpy as jnp
from jax import lax
from jax.experimental import pallas as pl
from jax.experimental.pallas import tpu as pltpu
from jax.experimental.pallas import tpu_sc as plsc

N_NODES = 10000
D = 128
N_EDGES = 320000

NC = 2
NS = 16
NW = NC * NS
B = 128
NIDX = 8
NROW = 2
NBLK = -(-N_EDGES // (NW * B * NIDX)) * NIDX
E_PAD = NW * NBLK * B
N_ACC = 10240
ROWS_PER_TILE_ACC = N_ACC // NS
NGRP = NBLK // NIDX


def _sc_body(idx_hbm, source_hbm, partial_hbm,
             idx0, idx1, idx2, idx3, idx4, idx5, idx6, idx7,
             rows0, rows1, zrow_v, acc_sh,
             isem0, isem1, isem2, isem3, isem4, isem5, isem6, isem7,
             gsem0, gsem1):
    idx_slots = (idx0, idx1, idx2, idx3, idx4, idx5, idx6, idx7)
    isems = (isem0, isem1, isem2, isem3, isem4, isem5, isem6, isem7)
    rows_bufs = (rows0, rows1)
    gsems = (gsem0, gsem1)
    c = lax.axis_index("c")
    s = lax.axis_index("s")
    wid = s * NC + c
    my_idx = idx_hbm.at[wid]

    zero = jnp.zeros((16,), jnp.float32)
    for i in range(16):
        for j in range(D // 16):
            zrow_v[i, pl.ds(j * 16, 16)] = zero

    acc_base = s * ROWS_PER_TILE_ACC

    def zbody(i, carry):
        pltpu.sync_copy(zrow_v, acc_sh.at[pl.ds(acc_base + i * 16, 16)])
        return carry

    lax.fori_loop(0, ROWS_PER_TILE_ACC // 16, zbody, 0)
    plsc.subcore_barrier()

    for k in range(NIDX):
        pltpu.async_copy(my_idx.at[k], idx_slots[k], isems[k])

    def wait_idx(k):
        pltpu.make_async_copy(my_idx.at[0], idx_slots[k], isems[k]).wait()

    def wait_rows(kr, kidx):
        pltpu.make_async_copy(source_hbm.at[idx_slots[kidx].at[0]],
                              rows_bufs[kr], gsems[kr]).wait()

    def finish_block(k, g):
        kp = (k - 1) % NROW
        ki = (k - 1) % NIDX
        pltpu.make_async_copy(source_hbm.at[idx_slots[ki].at[0]],
                              rows_bufs[kp], gsems[kp]).wait()
        pltpu.sync_copy(rows_bufs[kp], acc_sh.at[idx_slots[ki].at[1]],
                        add=True)
        p_next = g * NIDX + (k - 1) + NIDX
        if k == 0:
            pltpu.async_copy(my_idx.at[p_next], idx_slots[ki], isems[ki])
        else:
            @pl.when(p_next < NBLK)
            def _():
                pltpu.async_copy(my_idx.at[p_next], idx_slots[ki], isems[ki])

    def body(g, carry):
        for k in range(NIDX):
            j = g * NIDX + k
            wait_idx(k)
            pltpu.async_copy(source_hbm.at[idx_slots[k].at[0]],
                             rows_bufs[k % NROW], gsems[k % NROW])
            if k == 0:
                @pl.when(g > 0)
                def _():
                    finish_block(k, g)
            else:
                finish_block(k, g)
        return carry

    lax.fori_loop(0, NGRP, body, 0)
    kp = (NBLK - 1) % NROW
    ki = (NBLK - 1) % NIDX
    wait_rows(kp, ki)
    pltpu.sync_copy(rows_bufs[kp], acc_sh.at[idx_slots[ki].at[1]], add=True)
    plsc.subcore_barrier()

    pltpu.sync_copy(acc_sh.at[pl.ds(acc_base, ROWS_PER_TILE_ACC)],
                    partial_hbm.at[c].at[pl.ds(acc_base, ROWS_PER_TILE_ACC)])


_sc_partial = functools.partial(
    pl.kernel,
    out_type=jax.ShapeDtypeStruct((NC, N_ACC, D), jnp.float32),
    mesh=plsc.VectorSubcoreMesh(core_axis_name="c", subcore_axis_name="s"),
    scratch_types=(
        [pltpu.VMEM((2, B), jnp.int32) for _ in range(NIDX)] +
        [pltpu.VMEM((B, D), jnp.float32) for _ in range(NROW)] +
        [pltpu.VMEM((16, D), jnp.float32),
         pltpu.VMEM_SHARED((N_ACC, D), jnp.float32)] +
        [pltpu.SemaphoreType.DMA for _ in range(NIDX + NROW)]
    ),
)(_sc_body)


def _combine_body(t_ref, p0_ref, p1_ref, o_ref):
    o_ref[...] = t_ref[...] + p0_ref[...] + p1_ref[...]


def _combine(target, p0, p1):
    blk = 1000
    grid = N_NODES // blk
    spec = pl.BlockSpec((blk, D), lambda i: (i, 0))
    return pl.pallas_call(
        _combine_body,
        grid=(grid,),
        in_specs=[spec, spec, spec],
        out_specs=spec,
        out_shape=jax.ShapeDtypeStruct((N_NODES, D), jnp.float32),
    )(target, p0, p1)


@jax.jit
def kernel(source, target, edge_index):
    src = edge_index[0].astype(jnp.int32)
    dst = edge_index[1].astype(jnp.int32)
    pad = E_PAD - N_EDGES
    src_p = jnp.concatenate(
        [src, jnp.zeros((pad,), jnp.int32)]).reshape(NW, NBLK, B)
    dst_p = jnp.concatenate(
        [dst, jnp.full((pad,), N_NODES, jnp.int32)]).reshape(NW, NBLK, B)
    idx_p = jnp.stack([src_p, dst_p], axis=2)
    partial = _sc_partial(idx_p, source)
    return _combine(target, partial[0], partial[1])


# speedup vs baseline: 1.3840x; 1.3840x over previous
import functools

import jax
import jax.numpy as jnp
from jax import lax
from jax.experimental import pallas as pl
from jax.experimental.pallas import tpu as pltpu
from jax.experimental.pallas import tpu_sc as plsc

N_NODES = 10000
D = 128
N_EDGES = 320000

NC = 2
NS = 16
NW = NC * NS
B = 128
NBLK = -(-N_EDGES // (NW * B))
E_PAD = NW * NBLK * B
N_ACC = 10240
ROWS_PER_TILE_ACC = N_ACC // NS


def _sc_body(src_hbm, dst_hbm, source_hbm, partial_hbm,
             src_v, dst_v, rows_v, zrow_v, acc_sh, gsem):
    c = lax.axis_index("c")
    s = lax.axis_index("s")
    wid = s * NC + c

    pltpu.sync_copy(src_hbm.at[wid], src_v)
    pltpu.sync_copy(dst_hbm.at[wid], dst_v)

    zero = jnp.zeros((16,), jnp.float32)
    for i in range(16):
        for j in range(D // 16):
            zrow_v[i, pl.ds(j * 16, 16)] = zero

    acc_base = s * ROWS_PER_TILE_ACC

    def zbody(i, carry):
        pltpu.sync_copy(zrow_v, acc_sh.at[pl.ds(acc_base + i * 16, 16)])
        return carry

    lax.fori_loop(0, ROWS_PER_TILE_ACC // 16, zbody, 0)
    plsc.subcore_barrier()

    def body(j, carry):
        pltpu.async_copy(source_hbm.at[src_v.at[j]], rows_v, gsem).wait()
        pltpu.sync_copy(rows_v, acc_sh.at[dst_v.at[j]], add=True)
        return carry

    lax.fori_loop(0, NBLK, body, 0)
    plsc.subcore_barrier()

    pltpu.sync_copy(acc_sh.at[pl.ds(acc_base, ROWS_PER_TILE_ACC)],
                    partial_hbm.at[c].at[pl.ds(acc_base, ROWS_PER_TILE_ACC)])


_sc_partial = functools.partial(
    pl.kernel,
    out_type=jax.ShapeDtypeStruct((NC, N_ACC, D), jnp.float32),
    mesh=plsc.VectorSubcoreMesh(core_axis_name="c", subcore_axis_name="s"),
    scratch_types=[
        pltpu.VMEM((NBLK, B), jnp.int32),
        pltpu.VMEM((NBLK, B), jnp.int32),
        pltpu.VMEM((B, D), jnp.float32),
        pltpu.VMEM((16, D), jnp.float32),
        pltpu.VMEM_SHARED((N_ACC, D), jnp.float32),
        pltpu.SemaphoreType.DMA,
    ],
)(_sc_body)


def _combine_body(t_ref, p0_ref, p1_ref, o_ref):
    o_ref[...] = t_ref[...] + p0_ref[...] + p1_ref[...]


def _combine(target, p0, p1):
    blk = 1000
    grid = N_NODES // blk
    spec = pl.BlockSpec((blk, D), lambda i: (i, 0))
    return pl.pallas_call(
        _combine_body,
        grid=(grid,),
        in_specs=[spec, spec, spec],
        out_specs=spec,
        out_shape=jax.ShapeDtypeStruct((N_NODES, D), jnp.float32),
    )(target, p0, p1)


@jax.jit
def kernel(source, target, edge_index):
    src = edge_index[0].astype(jnp.int32)
    dst = edge_index[1].astype(jnp.int32)
    pad = E_PAD - N_EDGES
    src_p = jnp.concatenate(
        [src, jnp.zeros((pad,), jnp.int32)]).reshape(NW, NBLK, B)
    dst_p = jnp.concatenate(
        [dst, jnp.full((pad,), N_NODES, jnp.int32)]).reshape(NW, NBLK, B)
    partial = _sc_partial(src_p, dst_p, source)
    return _combine(target, partial[0], partial[1])
